# baseline (speedup 1.0000x reference)
import functools
import math

import jax
import jax.numpy as jnp
from jax import lax
from jax.experimental import pallas as pl
from jax.experimental.pallas import tpu as pltpu
from jax.experimental.pallas import tpu_sc as plsc

CHUNK = 128
RING = 4
_info = plsc.get_sparse_core_info()
NC, NS = _info.num_cores, _info.num_subcores
NW = NC * NS


@functools.lru_cache(maxsize=None)
def _make_sc_lookup(t_dim, bh_dim, vocab, d):
    scale = math.sqrt(d)
    dh_dim = d // 8
    db_dim = d // 16
    num_chunks = t_dim * bh_dim
    cpw = num_chunks // NW
    assert cpw % RING == 0 and cpw >= 2 * RING
    mesh = plsc.VectorSubcoreMesh(core_axis_name="c", subcore_axis_name="s")

    @functools.partial(
        pl.kernel,
        mesh=mesh,
        out_type=jax.ShapeDtypeStruct(
            (t_dim, dh_dim, bh_dim, 8, 128), jnp.float32
        ),
        scratch_types=[
            pltpu.VMEM((cpw, CHUNK), jnp.int32),
            pltpu.VMEM((RING, CHUNK, d), jnp.float32),
            pltpu.VMEM((RING, dh_dim, 8, 128), jnp.float32),
        ]
        + [pltpu.SemaphoreType.DMA] * (2 * RING),
        compiler_params=pltpu.CompilerParams(
            use_tc_tiling_on_sc=False, needs_layout_passes=False
        ),
    )
    def k(xg_hbm, tp_hbm, out_hbm, idx_v, gbuf, wbuf, *sems):
        gsem = sems[:RING]
        wsem = sems[RING:]
        wid = lax.axis_index("s") * NC + lax.axis_index("c")
        c0 = wid * cpw
        pltpu.sync_copy(xg_hbm.at[pl.ds(c0, cpw)], idx_v)

        lane = lax.iota(jnp.int32, 16)
        zeros16 = jnp.zeros((16,), jnp.int32)

        def start_gather(j, b):
            pltpu.async_copy(tp_hbm.at[idx_v.at[j]], gbuf.at[b], gsem[b])

        def gather_wait(j, b):
            pltpu.make_async_copy(
                tp_hbm.at[idx_v.at[j]], gbuf.at[b], gsem[b]
            ).wait()

        def transform(b):
            def gg_body(gg, carry):
                gg16 = gg * 16
                for db in range(db_dim):
                    for s in range(16):
                        colp = (lane + s) & 15
                        src = lane * d + gg16 * d + (colp + db * 16)
                        dst = (
                            ((colp + db * 16) << 7)
                            + lane
                            + gg16
                        )
                        v = plsc.load_gather(gbuf.at[b], [zeros16, src])
                        plsc.store_scatter(
                            wbuf.at[b], [zeros16, zeros16, dst], v * scale
                        )
                return carry

            lax.fori_loop(0, 8, gg_body, 0)

        def start_wb(j, b):
            c = c0 + j
            t = c // bh_dim
            bh = lax.rem(c, bh_dim)
            for dh in range(dh_dim):
                pltpu.async_copy(
                    wbuf.at[b, dh], out_hbm.at[t, dh, bh], wsem[b]
                )

        def wb_wait(b):
            for dh in range(dh_dim):
                pltpu.make_async_copy(
                    wbuf.at[b, dh], out_hbm.at[0, dh, 0], wsem[b]
                ).wait()

        for b in range(RING):
            start_gather(b, b)
        for b in range(RING):
            gather_wait(b, b)
            transform(b)
            start_wb(b, b)
            start_gather(RING + b, b)

        def group_body(g, carry):
            for b in range(RING):
                j = g * RING + b
                gather_wait(j, b)
                wb_wait(b)
                transform(b)
                start_wb(j, b)

                @pl.when(j + RING < cpw)
                def _():
                    start_gather(j + RING, b)

            return carry

        lax.fori_loop(1, cpw // RING, group_body, 0)
        for b in range(RING):
            wb_wait(b)

    return k


def kernel(x, table):
    b_dim, t_dim = x.shape
    vocab, d = table.shape
    bh_dim = b_dim // 128
    xg = jnp.swapaxes(x, 0, 1).astype(jnp.int32).reshape(t_dim * bh_dim, 128)
    out5 = _make_sc_lookup(t_dim, bh_dim, vocab, d)(xg, table)
    return out5.transpose(2, 4, 0, 1, 3).reshape(b_dim, t_dim, d)


# speedup vs baseline: 1.3058x; 1.3058x over previous
import functools
import math

import jax
import jax.numpy as jnp
from jax import lax
from jax.experimental import pallas as pl
from jax.experimental.pallas import tpu as pltpu
from jax.experimental.pallas import tpu_sc as plsc

CHUNK = 128
RING = 4
_info = plsc.get_sparse_core_info()
NC, NS = _info.num_cores, _info.num_subcores
NW = NC * NS


@functools.lru_cache(maxsize=None)
def _make_sc_lookup(t_dim, bh_dim, vocab, d):
    scale = math.sqrt(d)
    dh_dim = d // 8
    db_dim = d // 16
    num_chunks = t_dim * bh_dim
    cpw = num_chunks // NW
    assert cpw % RING == 0 and cpw >= 2 * RING
    mesh = plsc.VectorSubcoreMesh(core_axis_name="c", subcore_axis_name="s")

    @functools.partial(
        pl.kernel,
        mesh=mesh,
        out_type=jax.ShapeDtypeStruct(
            (t_dim, dh_dim, bh_dim, 8, 128), jnp.float32
        ),
        scratch_types=[
            pltpu.VMEM((cpw, CHUNK), jnp.int32),
            pltpu.VMEM((RING, CHUNK, d), jnp.float32),
            pltpu.VMEM((RING, dh_dim, 8, 128), jnp.float32),
        ]
        + [pltpu.SemaphoreType.DMA] * (2 * RING),
        compiler_params=pltpu.CompilerParams(
            use_tc_tiling_on_sc=False, needs_layout_passes=False
        ),
    )
    def k(xg_hbm, tp_hbm, out_hbm, idx_v, gbuf, wbuf, *sems):
        gsem = sems[:RING]
        wsem = sems[RING:]
        wid = lax.axis_index("s") * NC + lax.axis_index("c")
        c0 = wid * cpw
        pltpu.sync_copy(xg_hbm.at[pl.ds(c0, cpw)], idx_v)

        lane = lax.iota(jnp.int32, 16)
        zeros16 = jnp.zeros((16,), jnp.int32)

        def start_gather(j, b):
            pltpu.async_copy(tp_hbm.at[idx_v.at[j]], gbuf.at[b], gsem[b])

        def gather_wait(j, b):
            pltpu.make_async_copy(
                tp_hbm.at[idx_v.at[j]], gbuf.at[b], gsem[b]
            ).wait()

        def transform(b):
            def gg_body(gg, carry):
                gg16 = gg * 16
                for db in range(db_dim):
                    for s0 in range(0, 16, 8):
                        vs = []
                        for s in range(s0, s0 + 8):
                            colp = (lane + s) & 15
                            src = lane * d + gg16 * d + (colp + db * 16)
                            vs.append(
                                plsc.load_gather(gbuf.at[b], [zeros16, src])
                            )
                        for i, s in enumerate(range(s0, s0 + 8)):
                            colp = (lane + s) & 15
                            dst = ((colp + db * 16) << 7) + lane + gg16
                            plsc.store_scatter(
                                wbuf.at[b],
                                [zeros16, zeros16, dst],
                                vs[i] * scale,
                            )
                return carry

            lax.fori_loop(0, 8, gg_body, 0)

        def start_wb(j, b):
            c = c0 + j
            t = c // bh_dim
            bh = lax.rem(c, bh_dim)
            for dh in range(dh_dim):
                pltpu.async_copy(
                    wbuf.at[b, dh], out_hbm.at[t, dh, bh], wsem[b]
                )

        def wb_wait(b):
            for dh in range(dh_dim):
                pltpu.make_async_copy(
                    wbuf.at[b, dh], out_hbm.at[0, dh, 0], wsem[b]
                ).wait()

        for b in range(RING):
            start_gather(b, b)
        for b in range(RING):
            gather_wait(b, b)
            transform(b)
            start_wb(b, b)
            start_gather(RING + b, b)

        def group_body(g, carry):
            for b in range(RING):
                j = g * RING + b
                gather_wait(j, b)
                wb_wait(b)
                transform(b)
                start_wb(j, b)

                @pl.when(j + RING < cpw)
                def _():
                    start_gather(j + RING, b)

            return carry

        lax.fori_loop(1, cpw // RING, group_body, 0)
        for b in range(RING):
            wb_wait(b)

    return k


def kernel(x, table):
    b_dim, t_dim = x.shape
    vocab, d = table.shape
    bh_dim = b_dim // 128
    xg = jnp.swapaxes(x, 0, 1).astype(jnp.int32).reshape(t_dim * bh_dim, 128)
    out5 = _make_sc_lookup(t_dim, bh_dim, vocab, d)(xg, table)
    return out5.transpose(2, 4, 0, 1, 3).reshape(b_dim, t_dim, d)


# speedup vs baseline: 1.5109x; 1.1570x over previous
import functools
import math

import jax
import jax.numpy as jnp
from jax import lax
from jax.experimental import pallas as pl
from jax.experimental.pallas import tpu as pltpu
from jax.experimental.pallas import tpu_sc as plsc

CHUNK = 128
RING = 4
_info = plsc.get_sparse_core_info()
NC, NS = _info.num_cores, _info.num_subcores
NW = NC * NS


@functools.lru_cache(maxsize=None)
def _make_sc_lookup(t_dim, bh_dim, vocab, d):
    scale = math.sqrt(d)
    dh_dim = d // 8
    db_dim = d // 16
    num_chunks = t_dim * bh_dim
    cpw = num_chunks // NW
    assert cpw % RING == 0 and cpw >= 2 * RING
    mesh = plsc.VectorSubcoreMesh(core_axis_name="c", subcore_axis_name="s")

    @functools.partial(
        pl.kernel,
        mesh=mesh,
        out_type=jax.ShapeDtypeStruct(
            (t_dim, dh_dim, bh_dim, 8, 128), jnp.float32
        ),
        scratch_types=[
            pltpu.VMEM((cpw, CHUNK), jnp.int32),
            pltpu.VMEM((RING, CHUNK, d), jnp.float32),
            pltpu.VMEM((RING, dh_dim, 8, 128), jnp.float32),
        ]
        + [pltpu.SemaphoreType.DMA] * (2 * RING),
        compiler_params=pltpu.CompilerParams(
            use_tc_tiling_on_sc=False, needs_layout_passes=False
        ),
    )
    def k(xg_hbm, tp_hbm, out_hbm, idx_v, gbuf, wbuf, *sems):
        gsem = sems[:RING]
        wsem = sems[RING:]
        wid = lax.axis_index("s") * NC + lax.axis_index("c")
        c0 = wid * cpw
        pltpu.sync_copy(xg_hbm.at[pl.ds(c0, cpw)], idx_v)

        lane = lax.iota(jnp.int32, 16)
        zeros16 = jnp.zeros((16,), jnp.int32)

        def start_gather(j, b):
            pltpu.async_copy(tp_hbm.at[idx_v.at[j]], gbuf.at[b], gsem[b])

        def gather_wait(j, b):
            pltpu.make_async_copy(
                tp_hbm.at[idx_v.at[j]], gbuf.at[b], gsem[b]
            ).wait()

        def transform(b):
            def gg_body(gg, carry):
                gg16 = gg * 16
                for db in range(db_dim):
                    vs = []
                    for s in range(16):
                        colp = (lane + s) & 15
                        src = lane * d + gg16 * d + (colp + db * 16)
                        vs.append(
                            plsc.load_gather(gbuf.at[b], [zeros16, src])
                        )
                    for s in range(16):
                        colp = (lane + s) & 15
                        dst = ((colp + db * 16) << 7) + lane + gg16
                        plsc.store_scatter(
                            wbuf.at[b],
                            [zeros16, zeros16, dst],
                            vs[s] * scale,
                        )
                return carry

            lax.fori_loop(0, 8, gg_body, 0)

        def start_wb(j, b):
            c = c0 + j
            t = c // bh_dim
            bh = lax.rem(c, bh_dim)
            for dh in range(dh_dim):
                pltpu.async_copy(
                    wbuf.at[b, dh], out_hbm.at[t, dh, bh], wsem[b]
                )

        def wb_wait(b):
            for dh in range(dh_dim):
                pltpu.make_async_copy(
                    wbuf.at[b, dh], out_hbm.at[0, dh, 0], wsem[b]
                ).wait()

        for b in range(RING):
            start_gather(b, b)
        for b in range(RING):
            gather_wait(b, b)
            transform(b)
            start_wb(b, b)
            start_gather(RING + b, b)

        def group_body(g, carry):
            for b in range(RING):
                j = g * RING + b
                gather_wait(j, b)
                wb_wait(b)
                transform(b)
                start_wb(j, b)

                @pl.when(j + RING < cpw)
                def _():
                    start_gather(j + RING, b)

            return carry

        lax.fori_loop(1, cpw // RING, group_body, 0)
        for b in range(RING):
            wb_wait(b)

    return k


def kernel(x, table):
    b_dim, t_dim = x.shape
    vocab, d = table.shape
    bh_dim = b_dim // 128
    xg = jnp.swapaxes(x, 0, 1).astype(jnp.int32).reshape(t_dim * bh_dim, 128)
    out5 = _make_sc_lookup(t_dim, bh_dim, vocab, d)(xg, table)
    return out5.transpose(2, 4, 0, 1, 3).reshape(b_dim, t_dim, d)


# speedup vs baseline: 2.2694x; 1.5021x over previous
import functools
import math

import jax
import jax.numpy as jnp
from jax import lax
from jax.experimental import pallas as pl
from jax.experimental.pallas import tpu as pltpu
from jax.experimental.pallas import tpu_sc as plsc

CHUNK = 128
RING = 4
RING_A = 2
_info = plsc.get_sparse_core_info()
NC, NS = _info.num_cores, _info.num_subcores
NW = NC * NS


@functools.lru_cache(maxsize=None)
def _make_relayout(vocab, d):
    scale = math.sqrt(d)
    nblk = (vocab + 127) // 128
    vpad = nblk * 128
    base_iters = nblk // NW
    leftover = nblk - base_iters * NW
    assert leftover > 0 or vocab % 128 == 0
    db_dim = d // 16
    dh_dim = d // 8
    mesh = plsc.VectorSubcoreMesh(core_axis_name="c", subcore_axis_name="s")

    @functools.partial(
        pl.kernel,
        mesh=mesh,
        out_type=jax.ShapeDtypeStruct((vpad * d,), jnp.float32),
        scratch_types=[pltpu.VMEM((dh_dim, 8, 128), jnp.float32)] * RING_A
        + [pltpu.VMEM((128 * d,), jnp.float32)] * RING_A
        + [pltpu.SemaphoreType.DMA] * (2 * RING_A),
        compiler_params=pltpu.CompilerParams(needs_layout_passes=False),
    )
    def k(tt_hbm, tail_hbm, out_hbm, *refs):
        gbufs = refs[:RING_A]
        wbufs = refs[RING_A : 2 * RING_A]
        gsem = refs[2 * RING_A : 3 * RING_A]
        wsem = refs[3 * RING_A :]
        wid = lax.axis_index("s") * NC + lax.axis_index("c")
        lane = lax.iota(jnp.int32, 16)
        zeros16 = jnp.zeros((16,), jnp.int32)

        def start_read(ch, b):
            for dh in range(dh_dim):
                pltpu.async_copy(
                    tt_hbm.at[pl.ds(dh * 8, 8), pl.ds(ch * 128, 128)],
                    gbufs[b].at[dh],
                    gsem[b],
                )

        def read_wait(b):
            for dh in range(dh_dim):
                pltpu.make_async_copy(
                    tt_hbm.at[pl.ds(0, 8), pl.ds(0, 128)],
                    gbufs[b].at[dh],
                    gsem[b],
                ).wait()

        def transform(b):
            def rb_body(rb, carry):
                rb16 = rb * 16
                for db in range(db_dim):
                    vs = []
                    for s in range(16):
                        colp = (lane + s) & 15
                        src = (colp << 7) + (db * 2048) + rb16 + lane
                        vs.append(
                            plsc.load_gather(gbufs[b], [zeros16, zeros16, src])
                        )
                    for s in range(16):
                        colp = (lane + s) & 15
                        dst = (rb16 + lane) * d + db * 16 + colp
                        plsc.store_scatter(
                            wbufs[b], [dst], vs[s] * scale
                        )
                return carry

            lax.fori_loop(0, 8, rb_body, 0)

        def start_wb(ch, b):
            pltpu.async_copy(
                wbufs[b], out_hbm.at[pl.ds(ch * (128 * d), 128 * d)], wsem[b]
            )

        def wb_wait(b):
            pltpu.make_async_copy(
                wbufs[b], out_hbm.at[pl.ds(0, 128 * d)], wsem[b]
            ).wait()

        for b in range(RING_A):
            start_read(b * NW + wid, b)
        for b in range(RING_A):
            read_wait(b)
            transform(b)
            start_wb(b * NW + wid, b)
            start_read((RING_A + b) * NW + wid, b)

        def group_body(g, carry):
            for b in range(RING_A):
                j = g * RING_A + b
                read_wait(b)
                wb_wait(b)
                transform(b)
                start_wb(j * NW + wid, b)

                @pl.when(j + RING_A < base_iters)
                def _():
                    start_read((j + RING_A) * NW + wid, b)

            return carry

        lax.fori_loop(1, base_iters // RING_A, group_body, 0)
        for b in range(RING_A):
            wb_wait(b)

        for i in range(leftover):
            ch = base_iters * NW + i
            partial = ch * 128 + 128 > vocab

            @pl.when(wid == i)
            def _(ch=ch, partial=partial):
                if partial:
                    nvalid = (vocab - ch * 128) * d
                    pltpu.sync_copy(tail_hbm, wbufs[0].at[pl.ds(0, nvalid)])
                else:
                    for dh in range(dh_dim):
                        pltpu.sync_copy(
                            tt_hbm.at[pl.ds(dh * 8, 8), pl.ds(ch * 128, 128)],
                            gbufs[0].at[dh],
                        )
                    transform(0)
                pltpu.sync_copy(
                    wbufs[0], out_hbm.at[pl.ds(ch * (128 * d), 128 * d)]
                )

    return k


@functools.lru_cache(maxsize=None)
def _make_sc_lookup(t_dim, bh_dim, vpad, d):
    dh_dim = d // 8
    db_dim = d // 16
    num_chunks = t_dim * bh_dim
    cpw = num_chunks // NW
    assert cpw % RING == 0 and cpw >= 2 * RING
    mesh = plsc.VectorSubcoreMesh(core_axis_name="c", subcore_axis_name="s")

    @functools.partial(
        pl.kernel,
        mesh=mesh,
        out_type=jax.ShapeDtypeStruct(
            (t_dim, dh_dim, bh_dim, 8, 128), jnp.float32
        ),
        scratch_types=[
            pltpu.VMEM((cpw, CHUNK), jnp.int32),
            pltpu.VMEM((RING, CHUNK, d), jnp.float32),
            pltpu.VMEM((RING, dh_dim, 8, 128), jnp.float32),
        ]
        + [pltpu.SemaphoreType.DMA] * (2 * RING),
        compiler_params=pltpu.CompilerParams(
            use_tc_tiling_on_sc=False, needs_layout_passes=False
        ),
    )
    def k(xg_hbm, tp_hbm, out_hbm, idx_v, gbuf, wbuf, *sems):
        gsem = sems[:RING]
        wsem = sems[RING:]
        wid = lax.axis_index("s") * NC + lax.axis_index("c")
        c0 = wid * cpw
        pltpu.sync_copy(xg_hbm.at[pl.ds(c0, cpw)], idx_v)

        lane = lax.iota(jnp.int32, 16)
        zeros16 = jnp.zeros((16,), jnp.int32)

        def start_gather(j, b):
            pltpu.async_copy(tp_hbm.at[idx_v.at[j]], gbuf.at[b], gsem[b])

        def gather_wait(j, b):
            pltpu.make_async_copy(
                tp_hbm.at[idx_v.at[j]], gbuf.at[b], gsem[b]
            ).wait()

        def transform(b):
            def gg_body(gg, carry):
                gg16 = gg * 16
                for db in range(db_dim):
                    vs = []
                    for s in range(16):
                        colp = (lane + s) & 15
                        src = lane * d + gg16 * d + (colp + db * 16)
                        vs.append(
                            plsc.load_gather(gbuf.at[b], [zeros16, src])
                        )
                    for s in range(16):
                        colp = (lane + s) & 15
                        dst = ((colp + db * 16) << 7) + lane + gg16
                        plsc.store_scatter(
                            wbuf.at[b], [zeros16, zeros16, dst], vs[s]
                        )
                return carry

            lax.fori_loop(0, 8, gg_body, 0)

        def start_wb(j, b):
            c = c0 + j
            t = c // bh_dim
            bh = lax.rem(c, bh_dim)
            for dh in range(dh_dim):
                pltpu.async_copy(
                    wbuf.at[b, dh], out_hbm.at[t, dh, bh], wsem[b]
                )

        def wb_wait(b):
            for dh in range(dh_dim):
                pltpu.make_async_copy(
                    wbuf.at[b, dh], out_hbm.at[0, dh, 0], wsem[b]
                ).wait()

        for b in range(RING):
            start_gather(b, b)
        for b in range(RING):
            gather_wait(b, b)
            transform(b)
            start_wb(b, b)
            start_gather(RING + b, b)

        def group_body(g, carry):
            for b in range(RING):
                j = g * RING + b
                gather_wait(j, b)
                wb_wait(b)
                transform(b)
                start_wb(j, b)

                @pl.when(j + RING < cpw)
                def _():
                    start_gather(j + RING, b)

            return carry

        lax.fori_loop(1, cpw // RING, group_body, 0)
        for b in range(RING):
            wb_wait(b)

    return k


def kernel(x, table):
    b_dim, t_dim = x.shape
    vocab, d = table.shape
    bh_dim = b_dim // 128
    vpad = ((vocab + 127) // 128) * 128
    tt = jnp.swapaxes(table, 0, 1)
    nblk = vpad // 128
    if vocab % 128:
        tail = (table[(nblk - 1) * 128 :] * math.sqrt(d)).reshape(-1)
    else:
        tail = jnp.zeros((d,), jnp.float32)
    tlin = _make_relayout(vocab, d)(tt, tail).reshape(vpad, d)
    xg = jnp.swapaxes(x, 0, 1).astype(jnp.int32).reshape(t_dim * bh_dim, 128)
    out5 = _make_sc_lookup(t_dim, bh_dim, vpad, d)(xg, tlin)
    return out5.transpose(2, 4, 0, 1, 3).reshape(b_dim, t_dim, d)
